# baseline (speedup 1.0000x reference)
import functools

import jax
import jax.numpy as jnp
from jax import lax
from jax.experimental import pallas as pl
from jax.experimental.pallas import tpu as pltpu
from jax.experimental.pallas import tpu_sc as plsc

_N = 10000
_NP = 10240
_DIN = 128
_DH = 64
_NC = 2
_NS = 16
_B = 128
_NBUF = 4
_ROWS_PER_SUB = _NP // _NS


def _project(x, W, b):
    n, d = x.shape
    blk = 80
    nblk = n // blk
    npblk = _NP // blk

    def body(x_ref, w_ref, b_ref, o_ref):
        o_ref[...] = jnp.dot(x_ref[...], w_ref[0],
                             preferred_element_type=jnp.float32) + b_ref[0]

    w_s = W.reshape(d, _NC, _DH).transpose(1, 0, 2)
    b_s = b.reshape(_NC, 1, _DH)
    return pl.pallas_call(
        body,
        grid=(nblk, _NC),
        in_specs=[
            pl.BlockSpec((blk, d), lambda i, c: (i, 0)),
            pl.BlockSpec((1, d, _DH), lambda i, c: (c, 0, 0)),
            pl.BlockSpec((1, 1, _DH), lambda i, c: (c, 0, 0)),
        ],
        out_specs=pl.BlockSpec((blk, _DH), lambda i, c: (c * npblk + i, 0)),
        out_shape=jax.ShapeDtypeStruct((_NC * _NP, _DH), jnp.float32),
    )(x, w_s, b_s)


def _propagate(hs, srcb, dst3, zeros):
    k = srcb.shape[-2]
    mesh = plsc.VectorSubcoreMesh(core_axis_name="c", subcore_axis_name="s",
                                  num_cores=_NC, num_subcores=_NS)
    out_ty = jax.ShapeDtypeStruct((_NC * _NP, _DH), jnp.float32)

    @functools.partial(
        pl.kernel,
        out_type=(out_ty, out_ty),
        mesh=mesh,
        compiler_params=pltpu.CompilerParams(use_tc_tiling_on_sc=False),
        scratch_types=[
            pltpu.VMEM((k, _B), jnp.int32),
            pltpu.VMEM((k, _B), jnp.int32),
            pltpu.VMEM((_NBUF, _B, _DH), jnp.float32),
        ] + [pltpu.SemaphoreType.DMA] * (2 * _NBUF) + [
            pltpu.VMEM_SHARED((_NP, _DH), jnp.float32),
        ],
    )
    def run(hs_ref, srcb_ref, dst3_ref, z_ref, out_ref, t_ref,
            src_v, dst_v, rows, *sems_acc):
        gsem = sems_acc[:_NBUF]
        ssem = sems_acc[_NBUF:2 * _NBUF]
        acc = sems_acc[2 * _NBUF]
        cid = lax.axis_index("c")
        sid = lax.axis_index("s")
        pltpu.sync_copy(srcb_ref.at[cid, sid], src_v)
        pltpu.sync_copy(dst3_ref.at[sid], dst_v)
        r0 = sid * _ROWS_PER_SUB

        def one_round(tab_in, tab_out):
            pltpu.sync_copy(z_ref.at[pl.ds(r0, _ROWS_PER_SUB)],
                            acc.at[pl.ds(r0, _ROWS_PER_SUB)])
            plsc.subcore_barrier()

            for b in range(_NBUF):
                pltpu.async_copy(tab_in.at[src_v.at[b]], rows.at[b], gsem[b])

            @pl.loop(0, k // _NBUF)
            def _(i):
                j0 = _NBUF * i
                for b in range(_NBUF):
                    pltpu.make_async_copy(tab_in.at[src_v.at[j0 + b]],
                                          rows.at[b], gsem[b]).wait()
                    pltpu.async_copy(rows.at[b], acc.at[dst_v.at[j0 + b]],
                                     ssem[b], add=True)
                for b in range(_NBUF):
                    @pl.when(j0 + _NBUF + b < k)
                    def _(b=b, j0=j0):
                        pltpu.make_async_copy(rows.at[b],
                                              acc.at[dst_v.at[0]],
                                              ssem[b]).wait()
                        pltpu.async_copy(
                            tab_in.at[src_v.at[j0 + _NBUF + b]],
                            rows.at[b], gsem[b])

            for b in range(_NBUF):
                pltpu.make_async_copy(rows.at[b], acc.at[dst_v.at[0]],
                                      ssem[b]).wait()

            plsc.subcore_barrier()
            pltpu.sync_copy(
                acc.at[pl.ds(r0, _ROWS_PER_SUB)],
                tab_out.at[pl.ds(cid * _NP + r0, _ROWS_PER_SUB)])
            plsc.subcore_barrier()

        one_round(hs_ref, out_ref)
        one_round(out_ref, t_ref)
        one_round(t_ref, out_ref)

    return run(hs, srcb, dst3, zeros)


def kernel(x, edge_index, W, b):
    hs = _project(x, W, b)
    src = edge_index[0]
    dst = edge_index[1]
    e = src.shape[0]
    k = -(-e // (_NS * _B))
    k = -(-k // _NBUF) * _NBUF
    pad = _NS * _B * k - e
    src_p = jnp.concatenate([src, jnp.zeros((pad,), jnp.int32)])
    dst_p = jnp.concatenate([dst, jnp.full((pad,), _N, jnp.int32)])
    src3 = src_p.reshape(_NS, k, _B)
    dst3 = dst_p.reshape(_NS, k, _B)
    srcb = jnp.stack([src3, src3 + _NP])
    zeros = jnp.zeros((_NP, _DH), jnp.float32)
    out, _ = _propagate(hs, srcb, dst3, zeros)
    return jnp.concatenate([out[:_N], out[_NP:_NP + _N]], axis=1)


# speedup vs baseline: 1.7566x; 1.7566x over previous
import functools

import jax
import jax.numpy as jnp
from jax import lax
from jax.experimental import pallas as pl
from jax.experimental.pallas import tpu as pltpu
from jax.experimental.pallas import tpu_sc as plsc

_N = 10000
_NP = 10240
_DIN = 128
_DH = 64
_NC = 2
_NS = 16
_B = 128
_ROWS_PER_SUB = _NP // _NS


def _project(x, W, b):
    n, d = x.shape
    blk = 80
    nblk = n // blk
    npblk = _NP // blk

    def body(x_ref, w_ref, b_ref, o_ref):
        o_ref[...] = jnp.dot(x_ref[...], w_ref[0],
                             preferred_element_type=jnp.float32) + b_ref[0]

    w_s = W.reshape(d, _NC, _DH).transpose(1, 0, 2)
    b_s = b.reshape(_NC, 1, _DH)
    return pl.pallas_call(
        body,
        grid=(nblk, _NC),
        in_specs=[
            pl.BlockSpec((blk, d), lambda i, c: (i, 0)),
            pl.BlockSpec((1, d, _DH), lambda i, c: (c, 0, 0)),
            pl.BlockSpec((1, 1, _DH), lambda i, c: (c, 0, 0)),
        ],
        out_specs=pl.BlockSpec((blk, _DH), lambda i, c: (c * npblk + i, 0)),
        out_shape=jax.ShapeDtypeStruct((_NC * _NP, _DH), jnp.float32),
    )(x, w_s, b_s)


def _propagate(hs, pk3, zeros):
    k = pk3.shape[-2]
    mesh = plsc.VectorSubcoreMesh(core_axis_name="c", subcore_axis_name="s",
                                  num_cores=_NC, num_subcores=_NS)
    out_ty = jax.ShapeDtypeStruct((_NC * _NP, _DH), jnp.float32)

    @functools.partial(
        pl.kernel,
        out_type=out_ty,
        mesh=mesh,
        compiler_params=pltpu.CompilerParams(use_tc_tiling_on_sc=False),
        scratch_types=[
            pltpu.VMEM((k, _B), jnp.int32),
            pltpu.VMEM((2, _B), jnp.int32),
            pltpu.VMEM((2, _B), jnp.int32),
            pltpu.VMEM((_B, _DH), jnp.float32),
            pltpu.VMEM((_B, _DH), jnp.float32),
            pltpu.SemaphoreType.DMA,
            pltpu.SemaphoreType.DMA,
            pltpu.VMEM_SHARED((_NP, _DH), jnp.float32),
            pltpu.VMEM_SHARED((_NP, _DH), jnp.float32),
        ],
    )
    def run(hs_ref, pk3_ref, z_ref, out_ref,
            pk_v, src_b, dst_b, rows0, rows1, sem0, sem1, t0, t1):
        cid = lax.axis_index("c")
        sid = lax.axis_index("s")
        r0 = sid * _ROWS_PER_SUB
        rsl = pl.ds(r0, _ROWS_PER_SUB)
        pltpu.sync_copy(pk3_ref.at[sid], pk_v)
        pltpu.sync_copy(hs_ref.at[pl.ds(cid * _NP + r0, _ROWS_PER_SUB)],
                        t0.at[rsl])
        pltpu.sync_copy(z_ref, t1.at[rsl])
        plsc.subcore_barrier()

        def unpack(j, slot):
            @pl.loop(0, _B // 16)
            def _(c):
                v = pk_v[j, pl.ds(16 * c, 16)]
                src_b[slot, pl.ds(16 * c, 16)] = jnp.bitwise_and(v, 16383)
                dst_b[slot, pl.ds(16 * c, 16)] = lax.shift_right_logical(v, 14)

        def one_round(tab_in, tab_out):
            unpack(0, 0)
            pltpu.async_copy(tab_in.at[src_b.at[0]], rows0, sem0)

            @pl.loop(0, k // 2)
            def _(i):
                j = 2 * i
                unpack(j + 1, 1)
                pltpu.async_copy(tab_in.at[src_b.at[1]], rows1, sem1)
                pltpu.make_async_copy(tab_in.at[src_b.at[0]], rows0,
                                      sem0).wait()
                pltpu.sync_copy(rows0, tab_out.at[dst_b.at[0]], add=True)

                @pl.when(j + 2 < k)
                def _():
                    unpack(j + 2, 0)
                    pltpu.async_copy(tab_in.at[src_b.at[0]], rows0, sem0)

                pltpu.make_async_copy(tab_in.at[src_b.at[1]], rows1,
                                      sem1).wait()
                pltpu.sync_copy(rows1, tab_out.at[dst_b.at[1]], add=True)

            plsc.subcore_barrier()

        def zero(tab):
            pltpu.sync_copy(z_ref, tab.at[rsl])
            plsc.subcore_barrier()

        one_round(t0, t1)
        zero(t0)
        one_round(t1, t0)
        zero(t1)
        one_round(t0, t1)
        pltpu.sync_copy(t1.at[rsl],
                        out_ref.at[pl.ds(cid * _NP + r0, _ROWS_PER_SUB)])

    return run(hs, pk3, zeros)


def kernel(x, edge_index, W, b):
    hs = _project(x, W, b)
    src = edge_index[0]
    dst = edge_index[1]
    e = src.shape[0]
    k = -(-e // (_NS * _B))
    k += k % 2
    pad = _NS * _B * k - e
    src_p = jnp.concatenate([src, jnp.zeros((pad,), jnp.int32)])
    dst_p = jnp.concatenate([dst, jnp.full((pad,), _N, jnp.int32)])
    pk3 = (dst_p * 16384 + src_p).reshape(_NS, k, _B)
    zeros = jnp.zeros((_ROWS_PER_SUB, _DH), jnp.float32)
    out = _propagate(hs, pk3, zeros)
    return jnp.concatenate([out[:_N], out[_NP:_NP + _N]], axis=1)


# speedup vs baseline: 2.5031x; 1.4250x over previous
import functools

import jax
import jax.numpy as jnp
from jax import lax
from jax.experimental import pallas as pl
from jax.experimental.pallas import tpu as pltpu
from jax.experimental.pallas import tpu_sc as plsc

_N = 10000
_NP = 10240
_DIN = 128
_DH = 64
_NC = 2
_NS = 16
_B = 128
_ROWS_PER_SUB = _NP // _NS
_TAIL = _N - (_NS - 1) * _ROWS_PER_SUB


def _project(x, W, b):
    n, d = x.shape
    blk = 1000

    def body(x_ref, w_ref, b_ref, o_ref):
        o_ref[...] = jnp.dot(x_ref[...], w_ref[...],
                             preferred_element_type=jnp.float32) + b_ref[...]

    return pl.pallas_call(
        body,
        grid=(n // blk,),
        in_specs=[
            pl.BlockSpec((blk, d), lambda i: (i, 0)),
            pl.BlockSpec((d, d), lambda i: (0, 0)),
            pl.BlockSpec((1, d), lambda i: (0, 0)),
        ],
        out_specs=pl.BlockSpec((blk, d), lambda i: (i, 0)),
        out_shape=jax.ShapeDtypeStruct((n, d), jnp.float32),
    )(x, W, b.reshape(1, d))


def _propagate(hs, pk3, zeros):
    k = pk3.shape[-2]
    mesh = plsc.VectorSubcoreMesh(core_axis_name="c", subcore_axis_name="s",
                                  num_cores=_NC, num_subcores=_NS)
    out_ty = jax.ShapeDtypeStruct((_N, _DIN), jnp.float32)

    @functools.partial(
        pl.kernel,
        out_type=out_ty,
        mesh=mesh,
        compiler_params=pltpu.CompilerParams(use_tc_tiling_on_sc=False),
        scratch_types=[
            pltpu.VMEM((k, _B), jnp.int32),
            pltpu.VMEM((2, _B), jnp.int32),
            pltpu.VMEM((2, _B), jnp.int32),
            pltpu.VMEM((_B, _DH), jnp.float32),
            pltpu.VMEM((_B, _DH), jnp.float32),
            pltpu.SemaphoreType.DMA,
            pltpu.SemaphoreType.DMA,
            pltpu.SemaphoreType.DMA,
            pltpu.SemaphoreType.DMA,
            pltpu.VMEM_SHARED((_NP, _DH), jnp.float32),
            pltpu.VMEM_SHARED((_NP, _DH), jnp.float32),
        ],
    )
    def run(hs_ref, pk3_ref, z_ref, out_ref,
            pk_v, src_b, dst_b, rows0, rows1,
            semg0, semg1, sems0, sems1, t0, t1):
        cid = lax.axis_index("c")
        sid = lax.axis_index("s")
        r0 = sid * _ROWS_PER_SUB
        rsl = pl.ds(r0, _ROWS_PER_SUB)
        csl = pl.ds(cid * _DH, _DH)
        pltpu.sync_copy(pk3_ref.at[sid], pk_v)
        @pl.when(sid < _NS - 1)
        def _():
            pltpu.sync_copy(hs_ref.at[rsl, csl], t0.at[rsl])

        @pl.when(sid == _NS - 1)
        def _():
            pltpu.sync_copy(hs_ref.at[pl.ds(r0, _TAIL), csl],
                            t0.at[pl.ds(r0, _TAIL)])

        pltpu.sync_copy(z_ref, t1.at[rsl])
        plsc.subcore_barrier()

        def unpack(j, slot):
            @pl.loop(0, _B // 16)
            def _(c):
                v = pk_v[j, pl.ds(16 * c, 16)]
                src_b[slot, pl.ds(16 * c, 16)] = jnp.bitwise_and(v, 16383)
                dst_b[slot, pl.ds(16 * c, 16)] = lax.shift_right_logical(v, 14)

        def one_round(tab_in, tab_out):
            unpack(0, 0)
            pltpu.async_copy(tab_in.at[src_b.at[0]], rows0, semg0)

            @pl.loop(0, k // 2)
            def _(i):
                j = 2 * i
                pltpu.make_async_copy(tab_in.at[src_b.at[0]], rows0,
                                      semg0).wait()
                pltpu.async_copy(rows0, tab_out.at[dst_b.at[0]], sems0,
                                 add=True)

                @pl.when(i > 0)
                def _():
                    pltpu.make_async_copy(rows1, tab_out.at[dst_b.at[1]],
                                          sems1).wait()

                unpack(j + 1, 1)
                pltpu.async_copy(tab_in.at[src_b.at[1]], rows1, semg1)
                pltpu.make_async_copy(tab_in.at[src_b.at[1]], rows1,
                                      semg1).wait()
                pltpu.async_copy(rows1, tab_out.at[dst_b.at[1]], sems1,
                                 add=True)
                pltpu.make_async_copy(rows0, tab_out.at[dst_b.at[0]],
                                      sems0).wait()

                @pl.when(j + 2 < k)
                def _():
                    unpack(j + 2, 0)
                    pltpu.async_copy(tab_in.at[src_b.at[0]], rows0, semg0)

            pltpu.make_async_copy(rows1, tab_out.at[dst_b.at[1]],
                                  sems1).wait()
            plsc.subcore_barrier()

        def zero(tab):
            pltpu.sync_copy(z_ref, tab.at[rsl])
            plsc.subcore_barrier()

        one_round(t0, t1)
        zero(t0)
        one_round(t1, t0)
        zero(t1)
        one_round(t0, t1)

        @pl.when(sid < _NS - 1)
        def _():
            pltpu.sync_copy(t1.at[rsl], out_ref.at[rsl, csl])

        @pl.when(sid == _NS - 1)
        def _():
            pltpu.sync_copy(t1.at[pl.ds(r0, _TAIL)],
                            out_ref.at[pl.ds(r0, _TAIL), csl])

    return run(hs, pk3, zeros)


def kernel(x, edge_index, W, b):
    hs = _project(x, W, b)
    src = edge_index[0]
    dst = edge_index[1]
    e = src.shape[0]
    k = -(-e // (_NS * _B))
    k += k % 2
    pad = _NS * _B * k - e
    src_p = jnp.concatenate([src, jnp.zeros((pad,), jnp.int32)])
    dst_p = jnp.concatenate([dst, jnp.full((pad,), _N, jnp.int32)])
    pk3 = (dst_p * 16384 + src_p).reshape(_NS, k, _B)
    zeros = jnp.zeros((_ROWS_PER_SUB, _DH), jnp.float32)
    return _propagate(hs, pk3, zeros)


# speedup vs baseline: 2.5730x; 1.0279x over previous
import functools

import jax
import jax.numpy as jnp
from jax import lax
from jax.experimental import pallas as pl
from jax.experimental.pallas import tpu as pltpu
from jax.experimental.pallas import tpu_sc as plsc

_N = 10000
_NP = 10240
_DIN = 128
_DH = 64
_NC = 2
_NS = 16
_B = 192
_ROWS_PER_SUB = _NP // _NS
_TAIL = _N - (_NS - 1) * _ROWS_PER_SUB


def _project(x, W, b):
    n, d = x.shape
    blk = 1000

    def body(x_ref, w_ref, b_ref, o_ref):
        o_ref[...] = jnp.dot(x_ref[...], w_ref[...],
                             preferred_element_type=jnp.float32) + b_ref[...]

    return pl.pallas_call(
        body,
        grid=(n // blk,),
        in_specs=[
            pl.BlockSpec((blk, d), lambda i: (i, 0)),
            pl.BlockSpec((d, d), lambda i: (0, 0)),
            pl.BlockSpec((1, d), lambda i: (0, 0)),
        ],
        out_specs=pl.BlockSpec((blk, d), lambda i: (i, 0)),
        out_shape=jax.ShapeDtypeStruct((n, d), jnp.float32),
    )(x, W, b.reshape(1, d))


def _propagate(hs, pk3, zeros):
    k = pk3.shape[-2]
    mesh = plsc.VectorSubcoreMesh(core_axis_name="c", subcore_axis_name="s",
                                  num_cores=_NC, num_subcores=_NS)
    out_ty = jax.ShapeDtypeStruct((_N, _DIN), jnp.float32)

    @functools.partial(
        pl.kernel,
        out_type=out_ty,
        mesh=mesh,
        compiler_params=pltpu.CompilerParams(use_tc_tiling_on_sc=False),
        scratch_types=[
            pltpu.VMEM((k, _B), jnp.int32),
            pltpu.VMEM((2, _B), jnp.int32),
            pltpu.VMEM((2, _B), jnp.int32),
            pltpu.VMEM((_B, _DH), jnp.float32),
            pltpu.VMEM((_B, _DH), jnp.float32),
            pltpu.SemaphoreType.DMA,
            pltpu.SemaphoreType.DMA,
            pltpu.SemaphoreType.DMA,
            pltpu.SemaphoreType.DMA,
            pltpu.VMEM_SHARED((_NP, _DH), jnp.float32),
            pltpu.VMEM_SHARED((_NP, _DH), jnp.float32),
        ],
    )
    def run(hs_ref, pk3_ref, z_ref, out_ref,
            pk_v, src_b, dst_b, rows0, rows1,
            semg0, semg1, sems0, sems1, t0, t1):
        cid = lax.axis_index("c")
        sid = lax.axis_index("s")
        r0 = sid * _ROWS_PER_SUB
        rsl = pl.ds(r0, _ROWS_PER_SUB)
        csl = pl.ds(cid * _DH, _DH)
        pltpu.sync_copy(pk3_ref.at[sid], pk_v)
        @pl.when(sid < _NS - 1)
        def _():
            pltpu.sync_copy(hs_ref.at[rsl, csl], t0.at[rsl])

        @pl.when(sid == _NS - 1)
        def _():
            pltpu.sync_copy(hs_ref.at[pl.ds(r0, _TAIL), csl],
                            t0.at[pl.ds(r0, _TAIL)])

        pltpu.sync_copy(z_ref, t1.at[rsl])
        plsc.subcore_barrier()

        def unpack(j, slot):
            @pl.loop(0, _B // 16)
            def _(c):
                v = pk_v[j, pl.ds(16 * c, 16)]
                src_b[slot, pl.ds(16 * c, 16)] = jnp.bitwise_and(v, 16383)
                dst_b[slot, pl.ds(16 * c, 16)] = lax.shift_right_logical(v, 14)

        def one_round(tab_in, tab_out):
            unpack(0, 0)
            pltpu.async_copy(tab_in.at[src_b.at[0]], rows0, semg0)

            @pl.loop(0, k // 2)
            def _(i):
                j = 2 * i
                pltpu.make_async_copy(tab_in.at[src_b.at[0]], rows0,
                                      semg0).wait()
                pltpu.async_copy(rows0, tab_out.at[dst_b.at[0]], sems0,
                                 add=True)

                @pl.when(i > 0)
                def _():
                    pltpu.make_async_copy(rows1, tab_out.at[dst_b.at[1]],
                                          sems1).wait()

                unpack(j + 1, 1)
                pltpu.async_copy(tab_in.at[src_b.at[1]], rows1, semg1)
                pltpu.make_async_copy(tab_in.at[src_b.at[1]], rows1,
                                      semg1).wait()
                pltpu.async_copy(rows1, tab_out.at[dst_b.at[1]], sems1,
                                 add=True)
                pltpu.make_async_copy(rows0, tab_out.at[dst_b.at[0]],
                                      sems0).wait()

                @pl.when(j + 2 < k)
                def _():
                    unpack(j + 2, 0)
                    pltpu.async_copy(tab_in.at[src_b.at[0]], rows0, semg0)

            pltpu.make_async_copy(rows1, tab_out.at[dst_b.at[1]],
                                  sems1).wait()
            plsc.subcore_barrier()

        def zero(tab):
            pltpu.sync_copy(z_ref, tab.at[rsl])
            plsc.subcore_barrier()

        one_round(t0, t1)
        zero(t0)
        one_round(t1, t0)
        zero(t1)
        one_round(t0, t1)

        @pl.when(sid < _NS - 1)
        def _():
            pltpu.sync_copy(t1.at[rsl], out_ref.at[rsl, csl])

        @pl.when(sid == _NS - 1)
        def _():
            pltpu.sync_copy(t1.at[pl.ds(r0, _TAIL)],
                            out_ref.at[pl.ds(r0, _TAIL), csl])

    return run(hs, pk3, zeros)


def kernel(x, edge_index, W, b):
    hs = _project(x, W, b)
    src = edge_index[0]
    dst = edge_index[1]
    e = src.shape[0]
    k = -(-e // (_NS * _B))
    k += k % 2
    pad = _NS * _B * k - e
    src_p = jnp.concatenate([src, jnp.zeros((pad,), jnp.int32)])
    dst_p = jnp.concatenate([dst, jnp.full((pad,), _N, jnp.int32)])
    pk3 = (dst_p * 16384 + src_p).reshape(_NS, k, _B)
    zeros = jnp.zeros((_ROWS_PER_SUB, _DH), jnp.float32)
    return _propagate(hs, pk3, zeros)
